# Initial kernel scaffold; baseline (speedup 1.0000x reference)
import jax
import jax.numpy as jnp
from jax.experimental import pallas as pl


def kernel(s0, edge_index, T):
    raise NotImplementedError("write your pallas kernel here")


# speedup vs baseline: 6.9178x; 6.9178x over previous
import functools

import jax
import jax.numpy as jnp
from jax import lax
from jax.experimental import pallas as pl
from jax.experimental.pallas import tpu as pltpu
from jax.experimental.pallas import tpu_sc as plsc

N = 10000
E = 160000
NC = 2
NS = 16
NW = NC * NS
EPW = E // NW
LANES = 16
CHUNKS = (EPW + LANES - 1) // LANES
EBUF = CHUNKS * LANES
PBUF = N + LANES
SENT = N


def _sc_segment_last(s0f, ei):
  mesh = plsc.VectorSubcoreMesh(core_axis_name="c", subcore_axis_name="s")

  @functools.partial(
      pl.kernel,
      out_type=jax.ShapeDtypeStruct((NW * N,), jnp.int32),
      mesh=mesh,
      compiler_params=pltpu.CompilerParams(needs_layout_passes=False),
      scratch_types=[
          pltpu.VMEM((3 * N,), jnp.float32),
          pltpu.VMEM((EBUF,), jnp.int32),
          pltpu.VMEM((EBUF,), jnp.int32),
          pltpu.VMEM((PBUF,), jnp.int32),
      ],
  )
  def k(s0_hbm, ei_hbm, out_hbm, s0_v, src_v, dst_v, p_v):
    wid = lax.axis_index("s") * NC + lax.axis_index("c")
    base = wid * EPW
    pltpu.sync_copy(s0_hbm, s0_v)
    pltpu.sync_copy(ei_hbm.at[pl.ds(base, EPW)], src_v.at[pl.ds(0, EPW)])
    pltpu.sync_copy(ei_hbm.at[pl.ds(E + base, EPW)], dst_v.at[pl.ds(0, EPW)])

    lane = lax.iota(jnp.int32, LANES)
    minus1 = jnp.full((LANES,), -1, jnp.int32)

    def init_body(i, carry):
      p_v[pl.ds(i * LANES, LANES)] = minus1
      return carry

    lax.fori_loop(0, PBUF // LANES, init_body, 0)

    nxt = jnp.minimum(lane + 1, LANES - 1)

    def body(i, carry):
      off = i * LANES
      s_raw = src_v[pl.ds(off, LANES)]
      d_raw = dst_v[pl.ds(off, LANES)]
      valid = (off + lane) < EPW
      s = jnp.where(valid, s_raw, 0)
      d = jnp.where(valid, d_raw, SENT)
      g0 = plsc.load_gather(s0_v, [3 * s])
      g1 = plsc.load_gather(s0_v, [3 * s + 1])
      g2 = plsc.load_gather(s0_v, [3 * s + 2])
      bit = (jnp.maximum(g1, g2) > g0).astype(jnp.int32)
      packed = (base + off + lane) * 2 + bit
      key = d * LANES + lane
      ks, vs = plsc.sort_key_val(key, packed)
      dsort = lax.shift_right_logical(ks, 4)
      dnext = lax.gather(
          dsort, nxt[:, None],
          lax.GatherDimensionNumbers(
              offset_dims=(), collapsed_slice_dims=(0,),
              start_index_map=(0,)),
          slice_sizes=(1,),
          mode=lax.GatherScatterMode.PROMISE_IN_BOUNDS)
      wmask = jnp.logical_or(lane == LANES - 1, dsort != dnext)
      cur = plsc.load_gather(p_v, [dsort])
      newv = jnp.maximum(cur, vs)
      plsc.store_scatter(p_v, [dsort], newv, mask=wmask)
      return carry

    lax.fori_loop(0, CHUNKS, body, 0)
    pltpu.sync_copy(p_v.at[pl.ds(0, N)], out_hbm.at[pl.ds(wid * N, N)])

  return k(s0f, ei)


def _tc_update(pmat, sT, Tm):
  def body(p_ref, st_ref, tm_ref, out_ref):
    packed = jnp.max(p_ref[...], axis=0, keepdims=True)
    b = jnp.logical_and(packed >= 0, lax.bitwise_and(packed, 1) == 1)
    node = lax.broadcasted_iota(jnp.int32, (1, N), 1)
    odd = lax.bitwise_and(node, 1) == 1
    t = tm_ref[...]
    m = jnp.max(t, axis=1, keepdims=True)
    e = jnp.exp(t - m)
    sm = e / jnp.sum(e, axis=1, keepdims=True)
    st = st_ref[...]
    dn = (((0,), (0,)), ((), ()))
    hi = lax.Precision.HIGHEST
    a0 = lax.dot_general(sm[0:3, :], st, dn, precision=hi)
    a1 = lax.dot_general(sm[3:6, :], st, dn, precision=hi)
    a3 = lax.dot_general(sm[9:12, :], st, dn, precision=hi)
    out_ref[...] = jnp.where(b, jnp.where(odd, a1, a3), a0)

  return pl.pallas_call(
      body,
      out_shape=jax.ShapeDtypeStruct((3, N), jnp.float32),
  )(pmat, sT, Tm)


def kernel(s0, edge_index, T):
  s0f = s0.reshape(3 * N)
  pmat = _sc_segment_last(s0f, edge_index.reshape(2 * E)).reshape(NW, N)
  outT = _tc_update(pmat, s0.T, T.reshape(27, 3))
  return outT.T
